# baseline (speedup 1.0000x reference)
import functools

import jax
import jax.numpy as jnp
from jax import lax
from jax.experimental import pallas as pl
from jax.experimental.pallas import tpu as pltpu
from jax.experimental.pallas import tpu_sc as plsc

N = 10000
E = 320000
D = 128
LAYERS = 4
BN_EPS = 1e-5

NC = 2
NS = 16
NT = NC * NS
K = 80
PER_TILE = E // NT
NBUF = 3
CHUNKS = 126
GROUPS = CHUNKS // NBUF
UNROLL = 3
AGG_ROWS = N + 8
ROWS_PER_TILE = 624
ROWS_TAIL = N - NS * ROWS_PER_TILE

_mesh = plsc.VectorSubcoreMesh(core_axis_name="c", subcore_axis_name="s")


@functools.partial(
    pl.kernel,
    mesh=_mesh,
    out_type=jax.ShapeDtypeStruct((NC, N, D), jnp.float32),
    scratch_types=[
        pltpu.VMEM((UNROLL, NBUF, 2, K), jnp.int32),
        pltpu.VMEM((NBUF, K, D), jnp.float32),
        pltpu.VMEM_SHARED((AGG_ROWS, D), jnp.float32),
    ]
    + [pltpu.SemaphoreType.DMA] * (3 * NBUF),
)
def _sc_agg(h_hbm, idx_hbm, zeros_hbm, out_hbm,
            idx_v, rows_v, agg_sh, *sems):
    c = lax.axis_index("c")
    s = lax.axis_index("s")
    sem_i = sems[:NBUF]
    sem_g = sems[NBUF:2 * NBUF]
    sem_s = sems[2 * NBUF:]
    tid = c * NS + s

    r0 = s * ROWS_PER_TILE
    pltpu.sync_copy(zeros_hbm.at[pl.ds(0, ROWS_PER_TILE)],
                    agg_sh.at[pl.ds(r0, ROWS_PER_TILE)])

    @pl.when(s == NS - 1)
    def _zero_tail():
        rt = NS * ROWS_PER_TILE
        pltpu.sync_copy(zeros_hbm.at[pl.ds(0, ROWS_TAIL)],
                        agg_sh.at[pl.ds(rt, ROWS_TAIL)])

    plsc.subcore_barrier()

    def _idx_load(group, slot):
        return pltpu.make_async_copy(idx_hbm.at[tid, group], idx_v.at[slot],
                                     sem_i[slot])

    def _gather(slot, row, b):
        return pltpu.make_async_copy(h_hbm.at[idx_v.at[slot, row, 0]],
                                     rows_v.at[b], sem_g[b])

    def _scatter(slot, row, b):
        return pltpu.make_async_copy(rows_v.at[b],
                                     agg_sh.at[idx_v.at[slot, row, 1]],
                                     sem_s[b])

    _idx_load(0, 0).start()
    _idx_load(1, 1).start()
    _idx_load(0, 0).wait()
    _gather(0, 0, 0).start()
    _gather(0, 1, 1).start()
    for r in range(9):
        ch = r
        slot, row, b = (r // 3) % 3, r % 3, r % 3
        if r > 0:
            pr = r - 1
            _scatter((pr // 3) % 3, pr % 3, pr % 3).wait()
        if r % 3 == 0:
            _idx_load(r // 3 + 2, (slot + 2) % 3).start()
        _gather(slot, row, b).wait()
        _scatter(slot, row, b).start(add=True)
        r2 = r + 2
        slot2, row2, b2 = (r2 // 3) % 3, r2 % 3, r2 % 3
        if r2 % 3 == 0:
            _idx_load((r2 // 3) % 3, slot2).wait()
        _gather(slot2, row2, b2).start()

    def body(u, carry):
        last = GROUPS // UNROLL - 1
        for r in range(9):
            slot, row, b = (r // 3) % 3, r % 3, r % 3
            pr = (r - 1) % 9
            _scatter((pr // 3) % 3, pr % 3, pr % 3).wait()
            if r % 3 == 0:
                g2 = 3 * u + r // 3 + 2
                if r == 0:
                    _idx_load(g2, (slot + 2) % 3).start()
                else:
                    @pl.when(u < last)
                    def _():
                        _idx_load(g2, (slot + 2) % 3).start()
            _gather(slot, row, b).wait()
            _scatter(slot, row, b).start(add=True)
            r2 = r + 2
            slot2, row2, b2 = ((r2 % 9) // 3) % 3, r2 % 3, r2 % 3
            if r2 < 9:
                if r2 % 3 == 0:
                    _idx_load(0, slot2).wait()
                _gather(slot2, row2, b2).start()
            else:
                @pl.when(u < last)
                def _():
                    if r2 % 3 == 0:
                        _idx_load(0, slot2).wait()
                    _gather(slot2, row2, b2).start()
        return carry

    lax.fori_loop(1, GROUPS // UNROLL, body, 0)

    _scatter(((CHUNKS - 1) // 3) % 3, (CHUNKS - 1) % 3,
             (CHUNKS - 1) % 3).wait()

    plsc.subcore_barrier()
    pltpu.sync_copy(agg_sh.at[pl.ds(r0, ROWS_PER_TILE)],
                    out_hbm.at[c, pl.ds(r0, ROWS_PER_TILE)])

    @pl.when(s == NS - 1)
    def _tail():
        rt = NS * ROWS_PER_TILE
        pltpu.sync_copy(agg_sh.at[pl.ds(rt, ROWS_TAIL)],
                        out_hbm.at[c, pl.ds(rt, ROWS_TAIL)])


BLK = 1000


def _mlp_body(h_ref, a0_ref, a1_ref, w1_ref, s1_ref, t1_ref,
              w2_ref, s2_ref, t2_ref, o_ref, *, final_relu):
    z = h_ref[...] + a0_ref[...] + a1_ref[...]
    z = jnp.dot(z, w1_ref[...], preferred_element_type=jnp.float32)
    z = z * s1_ref[...] + t1_ref[...]
    z = jnp.maximum(z, 0.0)
    z = jnp.dot(z, w2_ref[...], preferred_element_type=jnp.float32)
    z = z * s2_ref[...] + t2_ref[...]
    if final_relu:
        z = jnp.maximum(z, 0.0)
    o_ref[...] = z


def _mlp(h, a0, a1, w1, s1, t1, w2, s2, t2, final_relu):
    row = lambda i: (i, 0)
    fixed = lambda i: (0, 0)
    return pl.pallas_call(
        functools.partial(_mlp_body, final_relu=final_relu),
        grid=(N // BLK,),
        in_specs=[
            pl.BlockSpec((BLK, D), row),
            pl.BlockSpec((BLK, D), row),
            pl.BlockSpec((BLK, D), row),
            pl.BlockSpec((D, D), fixed),
            pl.BlockSpec((1, D), fixed),
            pl.BlockSpec((1, D), fixed),
            pl.BlockSpec((D, D), fixed),
            pl.BlockSpec((1, D), fixed),
            pl.BlockSpec((1, D), fixed),
        ],
        out_specs=pl.BlockSpec((BLK, D), row),
        out_shape=jax.ShapeDtypeStruct((N, D), jnp.float32),
    )(h, a0, a1, w1, s1, t1, w2, s2, t2)


def kernel(x, edge_index, w1, b1, g1, be1, rm1, rv1, w2, b2, g2, be2, rm2, rv2):
    pad = CHUNKS * K - PER_TILE
    src = edge_index[0].astype(jnp.int32).reshape(NT, PER_TILE)
    dst = edge_index[1].astype(jnp.int32).reshape(NT, PER_TILE)
    src = jnp.concatenate(
        [src, jnp.zeros((NT, pad), jnp.int32)], axis=1)
    dst = jnp.concatenate(
        [dst, jnp.full((NT, pad), N, jnp.int32)], axis=1)
    src = src.reshape(NT, GROUPS, NBUF, K)
    dst = dst.reshape(NT, GROUPS, NBUF, K)
    idx = jnp.stack([src, dst], axis=3)
    s1 = g1 * lax.rsqrt(rv1 + BN_EPS)
    t1 = (b1 - rm1) * s1 + be1
    s2 = g2 * lax.rsqrt(rv2 + BN_EPS)
    t2 = (b2 - rm2) * s2 + be2
    zeros = jnp.zeros((ROWS_PER_TILE, D), jnp.float32)
    h = x.astype(jnp.float32)
    for l in range(LAYERS):
        parts = _sc_agg(h, idx, zeros)
        h = _mlp(h, parts[0], parts[1], w1[l],
                 s1[l][None, :], t1[l][None, :],
                 w2[l], s2[l][None, :], t2[l][None, :],
                 l < LAYERS - 1)
    return h
